# Initial kernel scaffold; baseline (speedup 1.0000x reference)
import jax
import jax.numpy as jnp
from jax.experimental import pallas as pl


def kernel(x, edge_index, batch, W1, b1, W2, b2, W3, b3, Wc, bc):
    raise NotImplementedError("write your pallas kernel here")


# speedup vs baseline: 11.9697x; 11.9697x over previous
import functools

import jax
import jax.numpy as jnp
from jax import lax
from jax.experimental import pallas as pl
from jax.experimental.pallas import tpu as pltpu
from jax.experimental.pallas import tpu_sc as plsc

N = 10000
E = 320000
IN_CH = 128
HID = 64
G = 64
TASKS = 2

NC, NS = 2, 16
NW = NC * NS
CHUNK = 128
CPT = -(-E // (NW * CHUNK))
E_PAD = NW * CHUNK * CPT
NPAD = 10240
TRASH = N
RPS = NPAD // NS
DW = 16

_P = jax.lax.Precision.HIGHEST
_mesh = plsc.VectorSubcoreMesh(core_axis_name="c", subcore_axis_name="s")
_SC_PARAMS = pltpu.CompilerParams(use_tc_tiling_on_sc=False)


def _deg_kernel(dst_hbm, ones_hbm, z_hbm, out_hbm, idxb, ones_v, dacc):
    cid = lax.axis_index("c")
    sid = lax.axis_index("s")
    wid = cid * NS + sid
    pltpu.sync_copy(z_hbm, dacc.at[pl.ds(sid * RPS, RPS)])
    pltpu.sync_copy(ones_hbm, ones_v)
    plsc.subcore_barrier()

    @pl.loop(0, CPT)
    def _(j):
        pltpu.sync_copy(dst_hbm.at[wid, j], idxb)
        pltpu.sync_copy(ones_v, dacc.at[idxb], add=True)

    plsc.subcore_barrier()
    pltpu.sync_copy(dacc.at[pl.ds(sid * RPS, RPS)],
                    out_hbm.at[cid, pl.ds(sid * RPS, RPS)])


def _deg_call(dstp, ones_blk, zeros_blk):
    return pl.kernel(
        _deg_kernel,
        out_type=jax.ShapeDtypeStruct((NC, NPAD, DW), jnp.float32),
        mesh=_mesh,
        compiler_params=_SC_PARAMS,
        scratch_types=[
            pltpu.VMEM((CHUNK,), jnp.int32),
            pltpu.VMEM((CHUNK, DW), jnp.float32),
            pltpu.VMEM_SHARED((NPAD, DW), jnp.float32),
        ],
    )(dstp, ones_blk, zeros_blk)


def _agg_kernel(hp_hbm, src_hbm, dst_hbm, z_hbm, out_hbm, idxb, rows, acc):
    cid = lax.axis_index("c")
    sid = lax.axis_index("s")
    wid = cid * NS + sid
    pltpu.sync_copy(z_hbm, acc.at[pl.ds(sid * RPS, RPS)])
    plsc.subcore_barrier()

    @pl.loop(0, CPT)
    def _(j):
        pltpu.sync_copy(src_hbm.at[wid, j], idxb.at[0])
        pltpu.sync_copy(dst_hbm.at[wid, j], idxb.at[1])
        pltpu.sync_copy(hp_hbm.at[idxb.at[0]], rows)
        pltpu.sync_copy(rows, acc.at[idxb.at[1]], add=True)

    plsc.subcore_barrier()
    pltpu.sync_copy(acc.at[pl.ds(sid * RPS, RPS)],
                    out_hbm.at[cid, pl.ds(sid * RPS, RPS)])


def _agg_call(hp, srcp, dstp, zeros_blk):
    return pl.kernel(
        _agg_kernel,
        out_type=jax.ShapeDtypeStruct((NC, NPAD, HID), jnp.float32),
        mesh=_mesh,
        compiler_params=_SC_PARAMS,
        scratch_types=[
            pltpu.VMEM((2, CHUNK), jnp.int32),
            pltpu.VMEM((CHUNK, HID), jnp.float32),
            pltpu.VMEM_SHARED((NPAD, HID), jnp.float32),
        ],
    )(hp, srcp, dstp, zeros_blk)


def _dinv(dp_ref):
    deg = dp_ref[0] + dp_ref[1] + 1.0
    return 1.0 / jnp.sqrt(deg[:, 0:1])


def _k1_body(x_ref, w_ref, dp_ref, hp_ref):
    hp_ref[...] = lax.dot_general(
        x_ref[...], w_ref[...], (((1,), (0,)), ((), ())), precision=_P
    ) * _dinv(dp_ref)


def _k2_body(p_ref, hp_ref, b_ref, dp_ref, w_ref, o_ref):
    dinv = _dinv(dp_ref)
    a = jnp.maximum(dinv * (p_ref[0] + p_ref[1] + hp_ref[...]) + b_ref[...], 0.0)
    o_ref[...] = lax.dot_general(
        a, w_ref[...], (((1,), (0,)), ((), ())), precision=_P
    ) * dinv


def _k4_body(p_ref, hp_ref, b_ref, dp_ref, batch_ref, wc_ref, bc_ref, o_ref):
    dinv = _dinv(dp_ref)
    a = jnp.maximum(dinv * (p_ref[0] + p_ref[1] + hp_ref[...]) + b_ref[...], 0.0)
    gid = lax.broadcasted_iota(jnp.int32, (NPAD, G), 1)
    oh = (batch_ref[...] == gid).astype(jnp.float32)
    sums = lax.dot_general(oh, a, (((0,), (0,)), ((), ())), precision=_P)
    cnts = lax.dot_general(oh, jnp.ones((NPAD, 1), jnp.float32),
                           (((0,), (0,)), ((), ())), precision=_P)
    pooled = sums / jnp.maximum(cnts, 1.0)
    o_ref[...] = lax.dot_general(
        pooled, wc_ref[...], (((1,), (0,)), ((), ())), precision=_P
    ) + bc_ref[...]


def kernel(x, edge_index, batch, W1, b1, W2, b2, W3, b3, Wc, bc):
    src = edge_index[0].astype(jnp.int32)
    dst = edge_index[1].astype(jnp.int32)
    srcp = jnp.concatenate(
        [src, jnp.zeros((E_PAD - E,), jnp.int32)]).reshape(NW, CPT, CHUNK)
    dstp = jnp.concatenate(
        [dst, jnp.full((E_PAD - E,), TRASH, jnp.int32)]).reshape(NW, CPT, CHUNK)
    xp = jnp.pad(x, ((0, NPAD - N), (0, 0)))
    batchp = jnp.pad(batch.astype(jnp.int32), (0, NPAD - N),
                     constant_values=G).reshape(NPAD, 1)
    z64 = jnp.zeros((RPS, HID), jnp.float32)
    zd = jnp.zeros((RPS, DW), jnp.float32)
    onesd = jnp.ones((CHUNK, DW), jnp.float32)
    b1r, b2r, b3r = b1.reshape(1, HID), b2.reshape(1, HID), b3.reshape(1, HID)
    bcr = bc.reshape(1, TASKS)

    dp = _deg_call(dstp, onesd, zd)

    hp1 = pl.pallas_call(
        _k1_body, out_shape=jax.ShapeDtypeStruct((NPAD, HID), jnp.float32),
    )(xp, W1, dp)

    p1 = _agg_call(hp1, srcp, dstp, z64)
    hp2 = pl.pallas_call(
        _k2_body, out_shape=jax.ShapeDtypeStruct((NPAD, HID), jnp.float32),
    )(p1, hp1, b1r, dp, W2)

    p2 = _agg_call(hp2, srcp, dstp, z64)
    hp3 = pl.pallas_call(
        _k2_body, out_shape=jax.ShapeDtypeStruct((NPAD, HID), jnp.float32),
    )(p2, hp2, b2r, dp, W3)

    p3 = _agg_call(hp3, srcp, dstp, z64)
    out = pl.pallas_call(
        _k4_body, out_shape=jax.ShapeDtypeStruct((G, TASKS), jnp.float32),
    )(p3, hp3, b3r, dp, batchp, Wc, bcr)
    return out


# speedup vs baseline: 16.1792x; 1.3517x over previous
import jax
import jax.numpy as jnp
from jax import lax
from jax.experimental import pallas as pl
from jax.experimental.pallas import tpu as pltpu
from jax.experimental.pallas import tpu_sc as plsc

N = 10000
E = 320000
IN_CH = 128
HID = 64
G = 64
TASKS = 2

NC, NS = 2, 16
NW = NC * NS
CHUNK = 128
NBUF = 8
DEPTH = 4
CPT = 80
E_PAD = NW * CHUNK * CPT
NPAD = 10240
TRASH = N
RPS = NPAD // NS
DW = 16

_P = jax.lax.Precision.HIGHEST
_mesh = plsc.VectorSubcoreMesh(core_axis_name="c", subcore_axis_name="s")
_SC_PARAMS = pltpu.CompilerParams(use_tc_tiling_on_sc=False)


def _deg_kernel(sd_hbm, ones_hbm, z_hbm, out_hbm, idx_all, ones_v, dacc, sem):
    cid = lax.axis_index("c")
    sid = lax.axis_index("s")
    wid = cid * NS + sid
    pltpu.sync_copy(z_hbm, dacc.at[pl.ds(sid * RPS, RPS)])
    pltpu.sync_copy(ones_hbm, ones_v)
    pltpu.sync_copy(sd_hbm.at[wid], idx_all)
    plsc.subcore_barrier()

    def s_desc(j, b):
        return pltpu.make_async_copy(
            ones_v, dacc.at[idx_all.at[j, 1]], sem.at[b])

    @pl.loop(0, CPT)
    def _(j):
        b = jnp.bitwise_and(j, NBUF - 1)

        @pl.when(j >= NBUF)
        def _():
            s_desc(j - NBUF, b).wait()

        s_desc(j, b).start(add=True)

    for b in range(NBUF):
        s_desc(CPT - NBUF + b, b).wait()
    plsc.subcore_barrier()
    pltpu.sync_copy(dacc.at[pl.ds(sid * RPS, RPS)],
                    out_hbm.at[cid, pl.ds(sid * RPS, RPS)])


def _deg_call(sd, ones_blk, zeros_blk):
    return pl.kernel(
        _deg_kernel,
        out_type=jax.ShapeDtypeStruct((NC, NPAD, DW), jnp.float32),
        mesh=_mesh,
        compiler_params=_SC_PARAMS,
        scratch_types=[
            pltpu.VMEM((CPT, 2, CHUNK), jnp.int32),
            pltpu.VMEM((CHUNK, DW), jnp.float32),
            pltpu.VMEM_SHARED((NPAD, DW), jnp.float32),
            pltpu.SemaphoreType.DMA((NBUF,)),
        ],
    )(sd, ones_blk, zeros_blk)


def _agg_kernel(hp_hbm, sd_hbm, z_hbm, out_hbm, idx_all, rows, acc, semg, sems):
    cid = lax.axis_index("c")
    sid = lax.axis_index("s")
    wid = cid * NS + sid
    pltpu.sync_copy(z_hbm, acc.at[pl.ds(sid * RPS, RPS)])
    pltpu.sync_copy(sd_hbm.at[wid], idx_all)
    plsc.subcore_barrier()

    def g_desc(j, b):
        return pltpu.make_async_copy(
            hp_hbm.at[idx_all.at[j, 0]], rows.at[b], semg.at[b])

    def s_desc(j, b):
        return pltpu.make_async_copy(
            rows.at[b], acc.at[idx_all.at[j, 1]], sems.at[b])

    @pl.loop(0, CPT + DEPTH)
    def _(j):
        @pl.when(j < CPT)
        def _():
            b = jnp.bitwise_and(j, NBUF - 1)

            @pl.when(j >= NBUF)
            def _():
                s_desc(j - NBUF, b).wait()

            g_desc(j, b).start()

        @pl.when(j >= DEPTH)
        def _():
            jd = j - DEPTH
            bd = jnp.bitwise_and(jd, NBUF - 1)
            g_desc(jd, bd).wait()
            s_desc(jd, bd).start(add=True)

    for b in range(NBUF):
        s_desc(CPT - NBUF + b, b).wait()
    plsc.subcore_barrier()
    pltpu.sync_copy(acc.at[pl.ds(sid * RPS, RPS)],
                    out_hbm.at[cid, pl.ds(sid * RPS, RPS)])


def _agg_call(hp, sd, zeros_blk):
    return pl.kernel(
        _agg_kernel,
        out_type=jax.ShapeDtypeStruct((NC, NPAD, HID), jnp.float32),
        mesh=_mesh,
        compiler_params=_SC_PARAMS,
        scratch_types=[
            pltpu.VMEM((CPT, 2, CHUNK), jnp.int32),
            pltpu.VMEM((NBUF, CHUNK, HID), jnp.float32),
            pltpu.VMEM_SHARED((NPAD, HID), jnp.float32),
            pltpu.SemaphoreType.DMA((NBUF,)),
            pltpu.SemaphoreType.DMA((NBUF,)),
        ],
    )(hp, sd, zeros_blk)


def _dinv(dp_ref):
    deg = dp_ref[0] + dp_ref[1] + 1.0
    return 1.0 / jnp.sqrt(deg[:, 0:1])


def _k1_body(x_ref, w_ref, dp_ref, hp_ref):
    hp_ref[...] = lax.dot_general(
        x_ref[...], w_ref[...], (((1,), (0,)), ((), ())), precision=_P
    ) * _dinv(dp_ref)


def _k2_body(p_ref, hp_ref, b_ref, dp_ref, w_ref, o_ref):
    dinv = _dinv(dp_ref)
    a = jnp.maximum(dinv * (p_ref[0] + p_ref[1] + hp_ref[...]) + b_ref[...], 0.0)
    o_ref[...] = lax.dot_general(
        a, w_ref[...], (((1,), (0,)), ((), ())), precision=_P
    ) * dinv


def _k4_body(p_ref, hp_ref, b_ref, dp_ref, batch_ref, wc_ref, bc_ref, o_ref):
    dinv = _dinv(dp_ref)
    a = jnp.maximum(dinv * (p_ref[0] + p_ref[1] + hp_ref[...]) + b_ref[...], 0.0)
    gid = lax.broadcasted_iota(jnp.int32, (NPAD, G), 1)
    oh = (batch_ref[...] == gid).astype(jnp.float32)
    sums = lax.dot_general(oh, a, (((0,), (0,)), ((), ())), precision=_P)
    cnts = lax.dot_general(oh, jnp.ones((NPAD, 1), jnp.float32),
                           (((0,), (0,)), ((), ())), precision=_P)
    pooled = sums / jnp.maximum(cnts, 1.0)
    o_ref[...] = lax.dot_general(
        pooled, wc_ref[...], (((1,), (0,)), ((), ())), precision=_P
    ) + bc_ref[...]


def kernel(x, edge_index, batch, W1, b1, W2, b2, W3, b3, Wc, bc):
    src = edge_index[0].astype(jnp.int32)
    dst = edge_index[1].astype(jnp.int32)
    srcg = jnp.concatenate(
        [src, jnp.zeros((E_PAD - E,), jnp.int32)]).reshape(NW, CPT, CHUNK)
    dstg = jnp.concatenate(
        [dst, jnp.full((E_PAD - E,), TRASH, jnp.int32)]).reshape(NW, CPT, CHUNK)
    sd = jnp.stack([srcg, dstg], axis=2)
    xp = jnp.pad(x, ((0, NPAD - N), (0, 0)))
    batchp = jnp.pad(batch.astype(jnp.int32), (0, NPAD - N),
                     constant_values=G).reshape(NPAD, 1)
    z64 = jnp.zeros((RPS, HID), jnp.float32)
    zd = jnp.zeros((RPS, DW), jnp.float32)
    onesd = jnp.ones((CHUNK, DW), jnp.float32)
    b1r, b2r, b3r = b1.reshape(1, HID), b2.reshape(1, HID), b3.reshape(1, HID)
    bcr = bc.reshape(1, TASKS)

    dp = _deg_call(sd, onesd, zd)

    hp1 = pl.pallas_call(
        _k1_body, out_shape=jax.ShapeDtypeStruct((NPAD, HID), jnp.float32),
    )(xp, W1, dp)

    p1 = _agg_call(hp1, sd, z64)
    hp2 = pl.pallas_call(
        _k2_body, out_shape=jax.ShapeDtypeStruct((NPAD, HID), jnp.float32),
    )(p1, hp1, b1r, dp, W2)

    p2 = _agg_call(hp2, sd, z64)
    hp3 = pl.pallas_call(
        _k2_body, out_shape=jax.ShapeDtypeStruct((NPAD, HID), jnp.float32),
    )(p2, hp2, b2r, dp, W3)

    p3 = _agg_call(hp3, sd, z64)
    out = pl.pallas_call(
        _k4_body, out_shape=jax.ShapeDtypeStruct((G, TASKS), jnp.float32),
    )(p3, hp3, b3r, dp, batchp, Wc, bcr)
    return out
